# baseline (speedup 1.0000x reference)
import functools

import jax
import jax.numpy as jnp
from jax import lax
from jax.experimental import pallas as pl
from jax.experimental.pallas import tpu as pltpu
from jax.experimental.pallas import tpu_sc as plsc

N_FIELDS = 3
B = 16384
D = 64
LANES = 16
NC = 2
NS = 16
NW = NC * NS
BPW = B // NW
CHUNK = 128
NCHUNK = BPW // CHUNK
NROWS = 1000000
TROWS = NROWS // 2
TBN = 2048


def _tc_transpose_body(tt_ref, out_ref):
    t = jnp.transpose(tt_ref[...], (1, 0))
    out_ref[...] = jnp.concatenate([t, t], axis=1)


_tc_transpose = pl.pallas_call(
    _tc_transpose_body,
    grid=((NROWS + TBN - 1) // TBN,),
    in_specs=[pl.BlockSpec((D, TBN), lambda j: (0, j))],
    out_specs=pl.BlockSpec((TBN, 2 * D), lambda j: (j, 0)),
    out_shape=jax.ShapeDtypeStruct((NROWS, 2 * D), jnp.float32),
)


def _gather_body(it0_hbm, it1_hbm, it2_hbm, tpad_hbm, out_hbm,
                 idx0, idx1, idx2,
                 r00, r01, r10, r11, r20, r21, out_v, sem0, sem1):
    items_hbm = (it0_hbm, it1_hbm, it2_hbm)
    idx_v = (idx0, idx1, idx2)
    rows_v = ((r00, r10, r20), (r01, r11, r21))
    sems = (sem0, sem1)
    wid = lax.axis_index("s") * NC + lax.axis_index("c")
    base = wid * BPW

    for f in range(N_FIELDS):
        pltpu.sync_copy(items_hbm[f].at[pl.ds(base, BPW)], idx_v[f])

    def fire(j, pb):
        return [
            pltpu.async_copy(
                tpad_hbm.at[idx_v[f].at[pl.ds(j * CHUNK, CHUNK)]],
                rows_v[pb][f],
                sems[pb],
            )
            for f in range(N_FIELDS)
        ]

    lane = lax.iota(jnp.int32, LANES)
    perms = [jnp.bitwise_xor(lane, 1 << t) for t in range(4)]
    masks = [lane == j for j in range(LANES)]
    dnums = lax.GatherDimensionNumbers(
        offset_dims=(), collapsed_slice_dims=(0,), start_index_map=(0,))

    def _shuffle(v, perm):
        return lax.gather(
            v, perm[:, None], dimension_numbers=dnums, slice_sizes=(1,),
            mode=lax.GatherScatterMode.PROMISE_IN_BOUNDS)

    copies = fire(0, 0)
    for j in range(NCHUNK):
        pb = j % 2
        nxt = fire(j + 1, 1 - pb) if j + 1 < NCHUNK else []
        for c in copies:
            c.wait()
        copies = nxt
        bufs = rows_v[pb]

        def grp_body(g, carry, _j=j, _bufs=bufs):
            vec = jnp.zeros((LANES,), jnp.float32)
            for jj in range(LANES):
                r = g * LANES + jj
                acc = None
                for k in range(D // LANES):
                    sl = pl.ds(k * LANES, LANES)
                    p = _bufs[0][r, sl] * _bufs[1][r, sl] * _bufs[2][r, sl]
                    acc = p if acc is None else acc + p
                for t in range(4):
                    acc = acc + _shuffle(acc, perms[t])
                vec = jnp.where(masks[jj], acc, vec)
            out_v[pl.ds(_j * CHUNK + g * LANES, LANES)] = (
                1.0 / (1.0 + jnp.exp(-vec)))
            return carry

        lax.fori_loop(0, CHUNK // LANES, grp_body, 0)

    pltpu.sync_copy(out_v, out_hbm.at[pl.ds(base, BPW)])


@functools.partial(
    pl.kernel,
    mesh=plsc.VectorSubcoreMesh(core_axis_name="c", subcore_axis_name="s"),
    out_type=jax.ShapeDtypeStruct((B,), jnp.float32),
    scratch_types=[
        pltpu.VMEM((BPW,), jnp.int32),
        pltpu.VMEM((BPW,), jnp.int32),
        pltpu.VMEM((BPW,), jnp.int32),
        pltpu.VMEM((CHUNK, 2 * D), jnp.float32),
        pltpu.VMEM((CHUNK, 2 * D), jnp.float32),
        pltpu.VMEM((CHUNK, 2 * D), jnp.float32),
        pltpu.VMEM((CHUNK, 2 * D), jnp.float32),
        pltpu.VMEM((CHUNK, 2 * D), jnp.float32),
        pltpu.VMEM((CHUNK, 2 * D), jnp.float32),
        pltpu.VMEM((BPW,), jnp.float32),
        pltpu.SemaphoreType.DMA,
        pltpu.SemaphoreType.DMA,
    ],
)
def _gather(it0_hbm, it1_hbm, it2_hbm, tpad_hbm, out_hbm,
            idx0, idx1, idx2,
            r00, r01, r10, r11, r20, r21, out_v, sem0, sem1):
    _gather_body(it0_hbm, it1_hbm, it2_hbm, tpad_hbm, out_hbm,
                 idx0, idx1, idx2,
                 r00, r01, r10, r11, r20, r21, out_v, sem0, sem1)


def kernel(items, item_table):
    tpad = _tc_transpose(item_table.T)
    return _gather(items[0], items[1], items[2], tpad)


# speedup vs baseline: 1.0531x; 1.0531x over previous
import functools

import jax
import jax.numpy as jnp
from jax import lax
from jax.experimental import pallas as pl
from jax.experimental.pallas import tpu as pltpu
from jax.experimental.pallas import tpu_sc as plsc

N_FIELDS = 3
B = 16384
D = 64
LANES = 16
NC = 2
NS = 16
NW = NC * NS
BPW = B // NW
CHUNK = 128
NCHUNK = BPW // CHUNK
NROWS = 1000000
TROWS = NROWS // 2
TBN = 2048


def _tc_transpose_body(tt_ref, out_ref):
    x = tt_ref[...]
    row = lax.broadcasted_iota(jnp.int32, (D, 2 * D), 0)
    col = lax.broadcasted_iota(jnp.int32, (D, 2 * D), 1)
    eye2 = (row == (col % D)).astype(jnp.float32)
    out_ref[...] = lax.dot_general(
        x, eye2, (((0,), (0,)), ((), ())),
        preferred_element_type=jnp.float32)


_tc_transpose = pl.pallas_call(
    _tc_transpose_body,
    grid=((NROWS + TBN - 1) // TBN,),
    in_specs=[pl.BlockSpec((D, TBN), lambda j: (0, j))],
    out_specs=pl.BlockSpec((TBN, 2 * D), lambda j: (j, 0)),
    out_shape=jax.ShapeDtypeStruct((NROWS, 2 * D), jnp.float32),
)


def _gather_body(it0_hbm, it1_hbm, it2_hbm, tpad_hbm, out_hbm,
                 idx0, idx1, idx2,
                 r00, r01, r10, r11, r20, r21, out_v, sem0, sem1):
    items_hbm = (it0_hbm, it1_hbm, it2_hbm)
    idx_v = (idx0, idx1, idx2)
    rows_v = ((r00, r10, r20), (r01, r11, r21))
    sems = (sem0, sem1)
    wid = lax.axis_index("s") * NC + lax.axis_index("c")
    base = wid * BPW

    for f in range(N_FIELDS):
        pltpu.sync_copy(items_hbm[f].at[pl.ds(base, BPW)], idx_v[f])

    def fire(j, pb):
        return [
            pltpu.async_copy(
                tpad_hbm.at[idx_v[f].at[pl.ds(j * CHUNK, CHUNK)]],
                rows_v[pb][f],
                sems[pb],
            )
            for f in range(N_FIELDS)
        ]

    lane = lax.iota(jnp.int32, LANES)
    perms = [jnp.bitwise_xor(lane, 1 << t) for t in range(4)]
    masks = [lane == j for j in range(LANES)]
    dnums = lax.GatherDimensionNumbers(
        offset_dims=(), collapsed_slice_dims=(0,), start_index_map=(0,))

    def _shuffle(v, perm):
        return lax.gather(
            v, perm[:, None], dimension_numbers=dnums, slice_sizes=(1,),
            mode=lax.GatherScatterMode.PROMISE_IN_BOUNDS)

    copies = fire(0, 0)
    for j in range(NCHUNK):
        pb = j % 2
        nxt = fire(j + 1, 1 - pb) if j + 1 < NCHUNK else []
        for c in copies:
            c.wait()
        copies = nxt
        bufs = rows_v[pb]

        def grp_body(g, carry, _j=j, _bufs=bufs):
            vec = jnp.zeros((LANES,), jnp.float32)
            for jj in range(LANES):
                r = g * LANES + jj
                acc = None
                for k in range(D // LANES):
                    sl = pl.ds(k * LANES, LANES)
                    p = _bufs[0][r, sl] * _bufs[1][r, sl] * _bufs[2][r, sl]
                    acc = p if acc is None else acc + p
                for t in range(4):
                    acc = acc + _shuffle(acc, perms[t])
                vec = jnp.where(masks[jj], acc, vec)
            out_v[pl.ds(_j * CHUNK + g * LANES, LANES)] = (
                1.0 / (1.0 + jnp.exp(-vec)))
            return carry

        lax.fori_loop(0, CHUNK // LANES, grp_body, 0)

    pltpu.sync_copy(out_v, out_hbm.at[pl.ds(base, BPW)])


@functools.partial(
    pl.kernel,
    mesh=plsc.VectorSubcoreMesh(core_axis_name="c", subcore_axis_name="s"),
    out_type=jax.ShapeDtypeStruct((B,), jnp.float32),
    scratch_types=[
        pltpu.VMEM((BPW,), jnp.int32),
        pltpu.VMEM((BPW,), jnp.int32),
        pltpu.VMEM((BPW,), jnp.int32),
        pltpu.VMEM((CHUNK, 2 * D), jnp.float32),
        pltpu.VMEM((CHUNK, 2 * D), jnp.float32),
        pltpu.VMEM((CHUNK, 2 * D), jnp.float32),
        pltpu.VMEM((CHUNK, 2 * D), jnp.float32),
        pltpu.VMEM((CHUNK, 2 * D), jnp.float32),
        pltpu.VMEM((CHUNK, 2 * D), jnp.float32),
        pltpu.VMEM((BPW,), jnp.float32),
        pltpu.SemaphoreType.DMA,
        pltpu.SemaphoreType.DMA,
    ],
)
def _gather(it0_hbm, it1_hbm, it2_hbm, tpad_hbm, out_hbm,
            idx0, idx1, idx2,
            r00, r01, r10, r11, r20, r21, out_v, sem0, sem1):
    _gather_body(it0_hbm, it1_hbm, it2_hbm, tpad_hbm, out_hbm,
                 idx0, idx1, idx2,
                 r00, r01, r10, r11, r20, r21, out_v, sem0, sem1)


def kernel(items, item_table):
    tpad = _tc_transpose(item_table.T)
    return _gather(items[0], items[1], items[2], tpad)


# speedup vs baseline: 1.7940x; 1.7036x over previous
import functools

import jax
import jax.numpy as jnp
from jax import lax
from jax.experimental import pallas as pl
from jax.experimental.pallas import tpu as pltpu
from jax.experimental.pallas import tpu_sc as plsc

N_FIELDS = 3
B = 16384
D = 64
LANES = 16
NC = 2
NS = 16
NW = NC * NS
BPW = B // NW
CHUNK = 128
NCHUNK = BPW // CHUNK
NROWS = 1000000
TROWS = NROWS // 2
TBN = 8192


def _tc_transpose_body(tt_ref, out_ref):
    x = tt_ref[...]
    row = lax.broadcasted_iota(jnp.int32, (D, D), 0)
    col = lax.broadcasted_iota(jnp.int32, (D, D), 1)
    eye = (row == col).astype(jnp.float32)
    t = lax.dot_general(
        x, eye, (((0,), (0,)), ((), ())),
        preferred_element_type=jnp.float32)
    out_ref[...] = jnp.concatenate(
        [t, jnp.zeros((TBN, D), jnp.float32)], axis=1)


_tc_transpose = pl.pallas_call(
    _tc_transpose_body,
    grid=((NROWS + TBN - 1) // TBN,),
    in_specs=[pl.BlockSpec((D, TBN), lambda j: (0, j))],
    out_specs=pl.BlockSpec((TBN, 2 * D), lambda j: (j, 0)),
    out_shape=jax.ShapeDtypeStruct((NROWS, 2 * D), jnp.float32),
)


def _gather_body(it0_hbm, it1_hbm, it2_hbm, tpad_hbm, out_hbm,
                 idx0, idx1, idx2,
                 r00, r01, r10, r11, r20, r21, out_v, sem0, sem1):
    items_hbm = (it0_hbm, it1_hbm, it2_hbm)
    idx_v = (idx0, idx1, idx2)
    rows_v = ((r00, r10, r20), (r01, r11, r21))
    sems = (sem0, sem1)
    wid = lax.axis_index("s") * NC + lax.axis_index("c")
    base = wid * BPW

    for f in range(N_FIELDS):
        pltpu.sync_copy(items_hbm[f].at[pl.ds(base, BPW)], idx_v[f])

    def fire(j, pb):
        return [
            pltpu.async_copy(
                tpad_hbm.at[idx_v[f].at[pl.ds(j * CHUNK, CHUNK)]],
                rows_v[pb][f],
                sems[pb],
            )
            for f in range(N_FIELDS)
        ]

    lane = lax.iota(jnp.int32, LANES)
    perms = [jnp.bitwise_xor(lane, 1 << t) for t in range(4)]
    masks = [lane == j for j in range(LANES)]
    dnums = lax.GatherDimensionNumbers(
        offset_dims=(), collapsed_slice_dims=(0,), start_index_map=(0,))

    def _shuffle(v, perm):
        return lax.gather(
            v, perm[:, None], dimension_numbers=dnums, slice_sizes=(1,),
            mode=lax.GatherScatterMode.PROMISE_IN_BOUNDS)

    copies = fire(0, 0)
    for j in range(NCHUNK):
        pb = j % 2
        nxt = fire(j + 1, 1 - pb) if j + 1 < NCHUNK else []
        for c in copies:
            c.wait()
        copies = nxt
        bufs = rows_v[pb]

        def grp_body(g, carry, _j=j, _bufs=bufs):
            vec = jnp.zeros((LANES,), jnp.float32)
            for jj in range(LANES):
                r = g * LANES + jj
                acc = None
                for k in range(D // LANES):
                    sl = pl.ds(k * LANES, LANES)
                    p = _bufs[0][r, sl] * _bufs[1][r, sl] * _bufs[2][r, sl]
                    acc = p if acc is None else acc + p
                for t in range(4):
                    acc = acc + _shuffle(acc, perms[t])
                vec = jnp.where(masks[jj], acc, vec)
            out_v[pl.ds(_j * CHUNK + g * LANES, LANES)] = (
                1.0 / (1.0 + jnp.exp(-vec)))
            return carry

        lax.fori_loop(0, CHUNK // LANES, grp_body, 0)

    pltpu.sync_copy(out_v, out_hbm.at[pl.ds(base, BPW)])


@functools.partial(
    pl.kernel,
    mesh=plsc.VectorSubcoreMesh(core_axis_name="c", subcore_axis_name="s"),
    out_type=jax.ShapeDtypeStruct((B,), jnp.float32),
    scratch_types=[
        pltpu.VMEM((BPW,), jnp.int32),
        pltpu.VMEM((BPW,), jnp.int32),
        pltpu.VMEM((BPW,), jnp.int32),
        pltpu.VMEM((CHUNK, 2 * D), jnp.float32),
        pltpu.VMEM((CHUNK, 2 * D), jnp.float32),
        pltpu.VMEM((CHUNK, 2 * D), jnp.float32),
        pltpu.VMEM((CHUNK, 2 * D), jnp.float32),
        pltpu.VMEM((CHUNK, 2 * D), jnp.float32),
        pltpu.VMEM((CHUNK, 2 * D), jnp.float32),
        pltpu.VMEM((BPW,), jnp.float32),
        pltpu.SemaphoreType.DMA,
        pltpu.SemaphoreType.DMA,
    ],
)
def _gather(it0_hbm, it1_hbm, it2_hbm, tpad_hbm, out_hbm,
            idx0, idx1, idx2,
            r00, r01, r10, r11, r20, r21, out_v, sem0, sem1):
    _gather_body(it0_hbm, it1_hbm, it2_hbm, tpad_hbm, out_hbm,
                 idx0, idx1, idx2,
                 r00, r01, r10, r11, r20, r21, out_v, sem0, sem1)


def kernel(items, item_table):
    tpad = _tc_transpose(item_table.T)
    return _gather(items[0], items[1], items[2], tpad)


# speedup vs baseline: 1.9529x; 1.0886x over previous
import functools

import jax
import jax.numpy as jnp
from jax import lax
from jax.experimental import pallas as pl
from jax.experimental.pallas import tpu as pltpu
from jax.experimental.pallas import tpu_sc as plsc

N_FIELDS = 3
B = 16384
D = 64
LANES = 16
NC = 2
NS = 16
NW = NC * NS
BPW = B // NW
CHUNK = 128
NCHUNK = BPW // CHUNK
NROWS = 1000000
TROWS = NROWS // 2
TBN = 16384


def _tc_transpose_body(tt_ref, out_ref):
    x = tt_ref[...]
    row = lax.broadcasted_iota(jnp.int32, (D, D), 0)
    col = lax.broadcasted_iota(jnp.int32, (D, D), 1)
    eye = (row == col).astype(jnp.float32)
    t = lax.dot_general(
        x, eye, (((0,), (0,)), ((), ())),
        preferred_element_type=jnp.float32)
    out_ref[...] = jnp.concatenate(
        [t, jnp.zeros((TBN, D), jnp.float32)], axis=1)


_tc_transpose = pl.pallas_call(
    _tc_transpose_body,
    grid=((NROWS + TBN - 1) // TBN,),
    in_specs=[pl.BlockSpec((D, TBN), lambda j: (0, j))],
    out_specs=pl.BlockSpec((TBN, 2 * D), lambda j: (j, 0)),
    out_shape=jax.ShapeDtypeStruct((NROWS, 2 * D), jnp.float32),
)


def _gather_body(it0_hbm, it1_hbm, it2_hbm, tpad_hbm, out_hbm,
                 idx0, idx1, idx2,
                 r00, r01, r10, r11, r20, r21, out_v, sem0, sem1):
    items_hbm = (it0_hbm, it1_hbm, it2_hbm)
    idx_v = (idx0, idx1, idx2)
    rows_v = ((r00, r10, r20), (r01, r11, r21))
    sems = (sem0, sem1)
    wid = lax.axis_index("s") * NC + lax.axis_index("c")
    base = wid * BPW

    for f in range(N_FIELDS):
        pltpu.sync_copy(items_hbm[f].at[pl.ds(base, BPW)], idx_v[f])

    def fire(j, pb):
        return [
            pltpu.async_copy(
                tpad_hbm.at[idx_v[f].at[pl.ds(j * CHUNK, CHUNK)]],
                rows_v[pb][f],
                sems[pb],
            )
            for f in range(N_FIELDS)
        ]

    lane = lax.iota(jnp.int32, LANES)
    perms = [jnp.bitwise_xor(lane, 1 << t) for t in range(4)]
    masks = [lane == j for j in range(LANES)]
    dnums = lax.GatherDimensionNumbers(
        offset_dims=(), collapsed_slice_dims=(0,), start_index_map=(0,))

    def _shuffle(v, perm):
        return lax.gather(
            v, perm[:, None], dimension_numbers=dnums, slice_sizes=(1,),
            mode=lax.GatherScatterMode.PROMISE_IN_BOUNDS)

    copies = fire(0, 0)
    for j in range(NCHUNK):
        pb = j % 2
        nxt = fire(j + 1, 1 - pb) if j + 1 < NCHUNK else []
        for c in copies:
            c.wait()
        copies = nxt
        bufs = rows_v[pb]

        def grp_body(g, carry, _j=j, _bufs=bufs):
            vec = jnp.zeros((LANES,), jnp.float32)
            for jj in range(LANES):
                r = g * LANES + jj
                acc = None
                for k in range(D // LANES):
                    sl = pl.ds(k * LANES, LANES)
                    p = _bufs[0][r, sl] * _bufs[1][r, sl] * _bufs[2][r, sl]
                    acc = p if acc is None else acc + p
                for t in range(4):
                    acc = acc + _shuffle(acc, perms[t])
                vec = jnp.where(masks[jj], acc, vec)
            out_v[pl.ds(_j * CHUNK + g * LANES, LANES)] = (
                1.0 / (1.0 + jnp.exp(-vec)))
            return carry

        lax.fori_loop(0, CHUNK // LANES, grp_body, 0)

    pltpu.sync_copy(out_v, out_hbm.at[pl.ds(base, BPW)])


@functools.partial(
    pl.kernel,
    mesh=plsc.VectorSubcoreMesh(core_axis_name="c", subcore_axis_name="s"),
    out_type=jax.ShapeDtypeStruct((B,), jnp.float32),
    scratch_types=[
        pltpu.VMEM((BPW,), jnp.int32),
        pltpu.VMEM((BPW,), jnp.int32),
        pltpu.VMEM((BPW,), jnp.int32),
        pltpu.VMEM((CHUNK, 2 * D), jnp.float32),
        pltpu.VMEM((CHUNK, 2 * D), jnp.float32),
        pltpu.VMEM((CHUNK, 2 * D), jnp.float32),
        pltpu.VMEM((CHUNK, 2 * D), jnp.float32),
        pltpu.VMEM((CHUNK, 2 * D), jnp.float32),
        pltpu.VMEM((CHUNK, 2 * D), jnp.float32),
        pltpu.VMEM((BPW,), jnp.float32),
        pltpu.SemaphoreType.DMA,
        pltpu.SemaphoreType.DMA,
    ],
)
def _gather(it0_hbm, it1_hbm, it2_hbm, tpad_hbm, out_hbm,
            idx0, idx1, idx2,
            r00, r01, r10, r11, r20, r21, out_v, sem0, sem1):
    _gather_body(it0_hbm, it1_hbm, it2_hbm, tpad_hbm, out_hbm,
                 idx0, idx1, idx2,
                 r00, r01, r10, r11, r20, r21, out_v, sem0, sem1)


def kernel(items, item_table):
    tpad = _tc_transpose(item_table.T)
    return _gather(items[0], items[1], items[2], tpad)


# speedup vs baseline: 2.1890x; 1.1209x over previous
import functools

import jax
import jax.numpy as jnp
from jax import lax
from jax.experimental import pallas as pl
from jax.experimental.pallas import tpu as pltpu
from jax.experimental.pallas import tpu_sc as plsc

N_FIELDS = 3
B = 16384
D = 64
LANES = 16
NC = 2
NS = 16
NW = NC * NS
BPW = B // NW
CHUNK = 128
NCHUNK = BPW // CHUNK
NROWS = 1000000
SPLIT = 524288
TBN = 16384
NBLK = SPLIT // TBN
MAXB = (NROWS + TBN - 1) // TBN - 1


def _tc_transpose_body(ta_ref, tb_ref, out_ref):
    row = lax.broadcasted_iota(jnp.int32, (D, D), 0)
    col = lax.broadcasted_iota(jnp.int32, (D, D), 1)
    eye = (row == col).astype(jnp.float32)

    def t(x):
        return lax.dot_general(
            x, eye, (((0,), (0,)), ((), ())),
            preferred_element_type=jnp.float32)

    out_ref[...] = jnp.concatenate([t(ta_ref[...]), t(tb_ref[...])], axis=1)


_tc_transpose = pl.pallas_call(
    _tc_transpose_body,
    grid=(NBLK,),
    in_specs=[
        pl.BlockSpec((D, TBN), lambda j: (0, j)),
        pl.BlockSpec((D, TBN), lambda j: (0, jnp.minimum(j + NBLK, MAXB))),
    ],
    out_specs=pl.BlockSpec((TBN, 2 * D), lambda j: (j, 0)),
    out_shape=jax.ShapeDtypeStruct((SPLIT, 2 * D), jnp.float32),
)


def _gather_body(it0_hbm, it1_hbm, it2_hbm, tpad_hbm, out_hbm,
                 idx0, idx1, idx2, pidx0, pidx1, pidx2,
                 r00, r01, r10, r11, r20, r21, out_v, sem0, sem1):
    items_hbm = (it0_hbm, it1_hbm, it2_hbm)
    idx_v = (idx0, idx1, idx2)
    pidx_v = (pidx0, pidx1, pidx2)
    rows_v = ((r00, r10, r20), (r01, r11, r21))
    sems = (sem0, sem1)
    wid = lax.axis_index("s") * NC + lax.axis_index("c")
    base = wid * BPW

    for f in range(N_FIELDS):
        pltpu.sync_copy(items_hbm[f].at[pl.ds(base, BPW)], idx_v[f])

    def pidx_body(t, carry):
        sl = pl.ds(t * LANES, LANES)
        for f in range(N_FIELDS):
            iv = idx_v[f][sl]
            pidx_v[f][sl] = iv - jnp.where(iv >= SPLIT, SPLIT, 0)
        return carry

    lax.fori_loop(0, BPW // LANES, pidx_body, 0)

    def fire(j, pb):
        return [
            pltpu.async_copy(
                tpad_hbm.at[pidx_v[f].at[pl.ds(j * CHUNK, CHUNK)]],
                rows_v[pb][f],
                sems[pb],
            )
            for f in range(N_FIELDS)
        ]

    lane = lax.iota(jnp.int32, LANES)
    perms = [jnp.bitwise_xor(lane, 1 << t) for t in range(4)]
    masks = [lane == j for j in range(LANES)]
    dnums = lax.GatherDimensionNumbers(
        offset_dims=(), collapsed_slice_dims=(0,), start_index_map=(0,))

    def _shuffle(v, perm):
        return lax.gather(
            v, perm[:, None], dimension_numbers=dnums, slice_sizes=(1,),
            mode=lax.GatherScatterMode.PROMISE_IN_BOUNDS)

    copies = fire(0, 0)
    for j in range(NCHUNK):
        pb = j % 2
        nxt = fire(j + 1, 1 - pb) if j + 1 < NCHUNK else []
        for c in copies:
            c.wait()
        copies = nxt
        bufs = rows_v[pb]

        def grp_body(g, carry, _j=j, _bufs=bufs):
            vec = jnp.zeros((LANES,), jnp.float32)
            gsl = pl.ds(_j * CHUNK + g * LANES, LANES)
            hoff = [
                jnp.where(idx_v[f][gsl] >= SPLIT, D, 0)
                for f in range(N_FIELDS)
            ]
            for jj in range(LANES):
                r = g * LANES + jj
                acc = None
                offs = [hoff[f][jj] for f in range(N_FIELDS)]
                for k in range(D // LANES):
                    p = None
                    for f in range(N_FIELDS):
                        v = _bufs[f][r, pl.ds(offs[f] + k * LANES, LANES)]
                        p = v if p is None else p * v
                    acc = p if acc is None else acc + p
                for t in range(4):
                    acc = acc + _shuffle(acc, perms[t])
                vec = jnp.where(masks[jj], acc, vec)
            out_v[pl.ds(_j * CHUNK + g * LANES, LANES)] = (
                1.0 / (1.0 + jnp.exp(-vec)))
            return carry

        lax.fori_loop(0, CHUNK // LANES, grp_body, 0)

    pltpu.sync_copy(out_v, out_hbm.at[pl.ds(base, BPW)])


@functools.partial(
    pl.kernel,
    mesh=plsc.VectorSubcoreMesh(core_axis_name="c", subcore_axis_name="s"),
    out_type=jax.ShapeDtypeStruct((B,), jnp.float32),
    scratch_types=[
        pltpu.VMEM((BPW,), jnp.int32),
        pltpu.VMEM((BPW,), jnp.int32),
        pltpu.VMEM((BPW,), jnp.int32),
        pltpu.VMEM((BPW,), jnp.int32),
        pltpu.VMEM((BPW,), jnp.int32),
        pltpu.VMEM((BPW,), jnp.int32),
        pltpu.VMEM((CHUNK, 2 * D), jnp.float32),
        pltpu.VMEM((CHUNK, 2 * D), jnp.float32),
        pltpu.VMEM((CHUNK, 2 * D), jnp.float32),
        pltpu.VMEM((CHUNK, 2 * D), jnp.float32),
        pltpu.VMEM((CHUNK, 2 * D), jnp.float32),
        pltpu.VMEM((CHUNK, 2 * D), jnp.float32),
        pltpu.VMEM((BPW,), jnp.float32),
        pltpu.SemaphoreType.DMA,
        pltpu.SemaphoreType.DMA,
    ],
)
def _gather(it0_hbm, it1_hbm, it2_hbm, tpad_hbm, out_hbm,
            idx0, idx1, idx2, pidx0, pidx1, pidx2,
            r00, r01, r10, r11, r20, r21, out_v, sem0, sem1):
    _gather_body(it0_hbm, it1_hbm, it2_hbm, tpad_hbm, out_hbm,
                 idx0, idx1, idx2, pidx0, pidx1, pidx2,
                 r00, r01, r10, r11, r20, r21, out_v, sem0, sem1)


def kernel(items, item_table):
    tpad = _tc_transpose(item_table.T, item_table.T)
    return _gather(items[0], items[1], items[2], tpad)
